# baseline (speedup 1.0000x reference)
import functools

import jax
import jax.numpy as jnp
from jax import lax
from jax.experimental import pallas as pl
from jax.experimental.pallas import tpu as pltpu
from jax.experimental.pallas import tpu_sc as plsc

N = 10000
E = 320000
H = 128
R = 8
B = 100
L = 16
VN = 32
VA = 16
G = N // B
NBLK = 10
BN = N // NBLK
GB = B // NBLK


def _tc1_body(nt_ref, emb_ref, wrel_ref, wroot_ref, b_ref, xr_ref, root_ref):
    nt = nt_ref[0]
    iot = lax.broadcasted_iota(jnp.int32, (VN, BN), 0)
    ohT = (iot == nt).astype(jnp.float32)
    emb = emb_ref[...]
    dn = (((0,), (0,)), ((), ()))
    for r in range(R):
        wp = jnp.dot(emb, wrel_ref[r], preferred_element_type=jnp.float32)
        xr_ref[r] = lax.dot_general(ohT, wp, dn, preferred_element_type=jnp.float32)
    wpr = jnp.dot(emb, wroot_ref[...], preferred_element_type=jnp.float32)
    root_ref[...] = (
        lax.dot_general(ohT, wpr, dn, preferred_element_type=jnp.float32) + b_ref[...]
    )


def _tc1(nodeTypes, emb_w, wrel, wroot, b):
    nt3 = nodeTypes.astype(jnp.int32).reshape(NBLK, 1, BN)
    return pl.pallas_call(
        _tc1_body,
        grid=(NBLK,),
        in_specs=[
            pl.BlockSpec((1, 1, BN), lambda i: (i, 0, 0)),
            pl.BlockSpec((VN, H), lambda i: (0, 0)),
            pl.BlockSpec((R, H, H), lambda i: (0, 0, 0)),
            pl.BlockSpec((H, H), lambda i: (0, 0)),
            pl.BlockSpec((1, H), lambda i: (0, 0)),
        ],
        out_specs=[
            pl.BlockSpec((R, BN, H), lambda i: (0, i, 0)),
            pl.BlockSpec((BN, H), lambda i: (i, 0)),
        ],
        out_shape=[
            jax.ShapeDtypeStruct((R, N, H), jnp.float32),
            jax.ShapeDtypeStruct((N, H), jnp.float32),
        ],
    )(nt3, emb_w, wrel, wroot, b.reshape(1, H))


def _tc2_body(root_ref, agg_ref, wrel_ref, wroot_ref, b_ref, xr_ref, root2_ref):
    x = jax.nn.relu(root_ref[...] + agg_ref[0] + agg_ref[1])
    for r in range(R):
        xr_ref[r] = jnp.dot(x, wrel_ref[r], preferred_element_type=jnp.float32)
    root2_ref[...] = (
        jnp.dot(x, wroot_ref[...], preferred_element_type=jnp.float32) + b_ref[...]
    )


def _tc2(root1, agg1, wrel, wroot, b):
    return pl.pallas_call(
        _tc2_body,
        grid=(NBLK,),
        in_specs=[
            pl.BlockSpec((BN, H), lambda i: (i, 0)),
            pl.BlockSpec((2, BN, H), lambda i: (0, i, 0)),
            pl.BlockSpec((R, H, H), lambda i: (0, 0, 0)),
            pl.BlockSpec((H, H), lambda i: (0, 0)),
            pl.BlockSpec((1, H), lambda i: (0, 0)),
        ],
        out_specs=[
            pl.BlockSpec((R, BN, H), lambda i: (0, i, 0)),
            pl.BlockSpec((BN, H), lambda i: (i, 0)),
        ],
        out_shape=[
            jax.ShapeDtypeStruct((R, N, H), jnp.float32),
            jax.ShapeDtypeStruct((N, H), jnp.float32),
        ],
    )(root1, agg1, wrel, wroot, b.reshape(1, H))


def _tc3_body(root_ref, agg_ref, seq_ref, emb_ref, hg_ref, mm_ref):
    embf = jax.nn.relu(root_ref[...] + agg_ref[0] + agg_ref[1])
    emb_ref[...] = embf
    dn = (((0,), (0,)), ((), ()))
    for g in range(GB):
        eg = embf[g * G:(g + 1) * G]
        hg_ref[0, g] = jnp.sum(eg, axis=0) * (1.0 / G)
        sg = seq_ref[0, g]
        mm_ref[0, g] = lax.dot_general(sg, eg, dn, preferred_element_type=jnp.float32)


def _tc3(root2, agg2, seq):
    seq4 = seq.reshape(NBLK, GB, G, L)
    return pl.pallas_call(
        _tc3_body,
        grid=(NBLK,),
        in_specs=[
            pl.BlockSpec((BN, H), lambda i: (i, 0)),
            pl.BlockSpec((2, BN, H), lambda i: (0, i, 0)),
            pl.BlockSpec((1, GB, G, L), lambda i: (i, 0, 0, 0)),
        ],
        out_specs=[
            pl.BlockSpec((BN, H), lambda i: (i, 0)),
            pl.BlockSpec((1, GB, H), lambda i: (i, 0, 0)),
            pl.BlockSpec((1, GB, L, H), lambda i: (i, 0, 0, 0)),
        ],
        out_shape=[
            jax.ShapeDtypeStruct((N, H), jnp.float32),
            jax.ShapeDtypeStruct((NBLK, GB, H), jnp.float32),
            jax.ShapeDtypeStruct((NBLK, GB, L, H), jnp.float32),
        ],
    )(root2, agg2, seq4)


def _tc4_body(hg_ref, mm_ref, act_in_ref, len_ref, embA_ref,
              wih_ref, whh_ref, bih_ref, bhh_ref,
              la_w_ref, la_b_ref, laf_w_ref, laf_b_ref,
              fl_w_ref, fl_b_ref, ff_w_ref, ff_b_ref,
              w1t_ref, lnb_ref,
              action_ref, final_ref, gp_ref):
    h_G = hg_ref[...]
    dn_t = (((1,), (1,)), ((), ()))

    ai = act_in_ref[...]
    iot = lax.broadcasted_iota(jnp.int32, (B, VA), 1)
    oh = (iot == ai).astype(jnp.float32)
    sos = jnp.dot(oh, embA_ref[...], preferred_element_type=jnp.float32)

    wih = wih_ref[...]
    whh = whh_ref[...]
    bih = bih_ref[...]
    bhh = bhh_ref[...]
    lens = len_ref[...]
    w1 = w1t_ref[...]
    lnb = lnb_ref[...]

    h = h_G
    for t in range(L):
        xt = sos if t == 0 else mm_ref[:, t - 1, :]
        gi = lax.dot_general(xt, wih, dn_t, preferred_element_type=jnp.float32) + bih
        gh = lax.dot_general(h, whh, dn_t, preferred_element_type=jnp.float32) + bhh
        i_r, i_z, i_n = gi[:, :H], gi[:, H:2 * H], gi[:, 2 * H:]
        h_r, h_z, h_n = gh[:, :H], gh[:, H:2 * H], gh[:, 2 * H:]
        rr = jax.nn.sigmoid(i_r + h_r)
        zz = jax.nn.sigmoid(i_z + h_z)
        nn_ = jnp.tanh(i_n + rr * h_n)
        h = (1.0 - zz) * nn_ + zz * h
        hm = h * (lens > t).astype(jnp.float32)
        gp_ref[:, t, :] = (
            lax.dot_general(hm, w1, dn_t, preferred_element_type=jnp.float32) + lnb
        )

    hr = jax.nn.relu(
        lax.dot_general(h_G, la_w_ref[...], dn_t, preferred_element_type=jnp.float32)
        + la_b_ref[...])
    action_ref[...] = (
        lax.dot_general(hr, laf_w_ref[...], dn_t, preferred_element_type=jnp.float32)
        + laf_b_ref[...])
    fr = jax.nn.relu(
        lax.dot_general(h_G, fl_w_ref[...], dn_t, preferred_element_type=jnp.float32)
        + fl_b_ref[...])
    s = jnp.sum(fr * ff_w_ref[...], axis=1, keepdims=True)
    final_ref[...] = jax.nn.sigmoid(s + ff_b_ref[...])


def _tc4(h_G, outmm, action_input, len_seq, emb_actions_w,
         gru_wih, gru_whh, gru_bih, gru_bhh,
         linAction_w, linAction_b, linActionF_w, linActionF_b,
         finLin_w, finLin_b, finF_w, finF_b, w1, linNodes_b):
    return pl.pallas_call(
        _tc4_body,
        out_shape=[
            jax.ShapeDtypeStruct((B, VA), jnp.float32),
            jax.ShapeDtypeStruct((B, 1), jnp.float32),
            jax.ShapeDtypeStruct((B, L, H), jnp.float32),
        ],
    )(h_G, outmm, action_input.astype(jnp.int32).reshape(B, 1),
      len_seq.astype(jnp.int32).reshape(B, 1), emb_actions_w,
      gru_wih, gru_whh, gru_bih.reshape(1, 3 * H), gru_bhh.reshape(1, 3 * H),
      linAction_w, linAction_b.reshape(1, H), linActionF_w,
      linActionF_b.reshape(1, VA),
      finLin_w, finLin_b.reshape(1, H), finF_w,
      jnp.broadcast_to(finF_b.reshape(1, 1), (B, 1)),
      w1, linNodes_b.reshape(1, H))


def _tc5_body(emb_ref, gp_ref, w2_ref, f_ref, out_ref):
    w2 = w2_ref[...]
    f = f_ref[...]
    dn_t = (((1,), (1,)), ((), ()))
    for g in range(GB):
        eg = emb_ref[...][g * G:(g + 1) * G]
        npart = lax.dot_general(eg, w2, dn_t, preferred_element_type=jnp.float32)
        gp = gp_ref[g]
        t3 = jax.nn.relu(npart[:, None, :] + gp[None, :, :])
        logits = jnp.sum(t3 * f[0][None, None, :], axis=-1)
        m = jnp.max(logits, axis=0, keepdims=True)
        e = jnp.exp(logits - m)
        s = jnp.sum(e, axis=0, keepdims=True)
        out_ref[0, g] = e / s


def _tc5(nodeEmb, gru_part, w2, f):
    return pl.pallas_call(
        _tc5_body,
        grid=(NBLK,),
        in_specs=[
            pl.BlockSpec((BN, H), lambda i: (i, 0)),
            pl.BlockSpec((GB, L, H), lambda i: (i, 0, 0)),
            pl.BlockSpec((H, H), lambda i: (0, 0)),
            pl.BlockSpec((1, H), lambda i: (0, 0)),
        ],
        out_specs=[pl.BlockSpec((1, GB, G, L), lambda i: (i, 0, 0, 0))],
        out_shape=[jax.ShapeDtypeStruct((NBLK, GB, G, L), jnp.float32)],
    )(nodeEmb, gru_part, w2, f)[0]


NC = 2
NS = 16
NW = NC * NS
LN = 16
GSZ = 80
NGRP = GSZ // LN
GSA = 128
NGA = GSA // LN
EC32P = 10112
NGP = EC32P // GSA
EROWS = E // GSZ
EC16 = E // NS
EC32 = E // NW
CH = 25
NSTR = N // NS

@functools.lru_cache(maxsize=None)
def _sc_mesh():
    return plsc.VectorSubcoreMesh(core_axis_name="c", subcore_axis_name="s",
                                  num_cores=NC, num_subcores=NS)


def _iota16():
    return lax.iota(jnp.int32, LN)


def _stripe(s):
    base = s * 624
    length = 624 if s < NS - 1 else N - 624 * (NS - 1)
    return base, length


TROWS = N * R // H
NG = EC32 // GSZ


def _sc_count_body(dst_hbm, et_hbm, out_hbm, tab, dstb, etb):
    c = lax.axis_index("c")
    s = lax.axis_index("s")
    wid = s * NC + c
    zer = jnp.zeros((LN,), jnp.float32)

    def _zt(i, _):
        for kk in range(H // LN):
            tab[i, pl.ds(kk * LN, LN)] = zer
        return 0
    lax.fori_loop(0, TROWS, _zt, 0)

    w0 = wid * EC32
    pltpu.sync_copy(dst_hbm.at[pl.ds(w0, EC32)], dstb)
    pltpu.sync_copy(et_hbm.at[pl.ds(w0, EC32)], etb)
    ones = jnp.ones((LN,), jnp.float32)

    def _grp(i, _):
        idx16 = (dstb[pl.ds(i * LN, LN)] * R + etb[pl.ds(i * LN, LN)])
        plsc.addupdate_scatter(tab, [idx16 >> 7, idx16 & (H - 1)], ones)
        return 0
    lax.fori_loop(0, EC32 // LN, _grp, 0)
    pltpu.sync_copy(tab, out_hbm.at[wid])


@functools.lru_cache(maxsize=None)
def _sc_count_call():
    return pl.kernel(
        _sc_count_body,
        out_type=jax.ShapeDtypeStruct((NW, TROWS, H), jnp.float32),
        mesh=_sc_mesh(),
        compiler_params=pltpu.CompilerParams(needs_layout_passes=False),
        scratch_types=[
            pltpu.VMEM((TROWS, H), jnp.float32),
            pltpu.VMEM((EC32,), jnp.int32),
            pltpu.VMEM((EC32,), jnp.int32),
        ],
    )


def _tc_rcp_body(cnt_ref, src_ref, et_ref, rcp_ref, gidx_ref):
    tot = jnp.sum(cnt_ref[...], axis=0)
    rcp_ref[...] = 1.0 / jnp.maximum(tot, 1.0)
    gidx_ref[...] = et_ref[...] * N + src_ref[...]


def _tc_rcp(cnt_parts, src1d, et1d):
    rcp, gidx = pl.pallas_call(
        _tc_rcp_body,
        out_shape=[
            jax.ShapeDtypeStruct((TROWS, H), jnp.float32),
            jax.ShapeDtypeStruct((E // H, H), jnp.int32),
        ],
    )(cnt_parts, src1d.reshape(E // H, H), et1d.reshape(E // H, H))
    return rcp, gidx.reshape(E)


def _sc_scale2_body(rcp_hbm, dst_hbm, et_hbm, scale_hbm, tab, dstb, etb, scb):
    c = lax.axis_index("c")
    s = lax.axis_index("s")
    wid = s * NC + c
    pltpu.sync_copy(rcp_hbm, tab)
    w0 = wid * EC32
    pltpu.sync_copy(dst_hbm.at[pl.ds(w0, EC32)], dstb)
    pltpu.sync_copy(et_hbm.at[pl.ds(w0, EC32)], etb)

    def _grp(i, _):
        idx16 = (dstb[pl.ds(i * LN, LN)] * R + etb[pl.ds(i * LN, LN)])
        scb[pl.ds(i * LN, LN)] = plsc.load_gather(
            tab, [idx16 >> 7, idx16 & (H - 1)])
        return 0
    lax.fori_loop(0, EC32 // LN, _grp, 0)
    pltpu.sync_copy(scb, scale_hbm.at[pl.ds(w0, EC32)])


@functools.lru_cache(maxsize=None)
def _sc_scale2_call():
    return pl.kernel(
        _sc_scale2_body,
        out_type=jax.ShapeDtypeStruct((E,), jnp.float32),
        mesh=_sc_mesh(),
        compiler_params=pltpu.CompilerParams(needs_layout_passes=False),
        scratch_types=[
            pltpu.VMEM((TROWS, H), jnp.float32),
            pltpu.VMEM((EC32,), jnp.int32),
            pltpu.VMEM((EC32,), jnp.int32),
            pltpu.VMEM((EC32,), jnp.float32),
        ],
    )


def _edge_scale(src1d, dst1d, et1d):
    cnt_parts = _sc_count_call()(dst1d, et1d)
    rcp, gidx1d = _tc_rcp(cnt_parts, src1d, et1d)
    return _sc_scale2_call()(rcp, dst1d, et1d), gidx1d


def _sc_agg_body(xr_hbm, gidx_hbm, dst_hbm, sc_hbm, out_hbm,
                 acc_sh, gsc,
                 gidxa, didxa, scla, gidxb, didxb, sclb, rowsa, rowsb,
                 sema, semb, dsema, dsemb):
    c = lax.axis_index("c")
    s = lax.axis_index("s")
    wid = s * NC + c

    def _zr(i, _):
        for kk in range(H // LN):
            rowsa[i, pl.ds(kk * LN, LN)] = jnp.zeros((LN,), jnp.float32)
        return 0
    lax.fori_loop(0, GSA, _zr, 0)
    sb = s * 624
    for j in range(5):
        sz = GSA if j < 4 else 112
        pltpu.sync_copy(rowsa.at[pl.ds(0, sz)],
                        acc_sh.at[pl.ds(sb + j * GSA, sz)])

    @pl.when(s == NS - 1)
    def _():
        pltpu.sync_copy(rowsa, acc_sh.at[pl.ds(N - GSA, GSA)])

    plsc.subcore_barrier()

    w0 = wid * EC32P
    pltpu.sync_copy(gidx_hbm.at[pl.ds(w0, EC32P)], gsc)

    def _fire(g, gidx, didx, scl, sem, dsem):
        for k in range(NGA):
            gidx[pl.ds(k * LN, LN)] = gsc[pl.ds(g * GSA + k * LN, LN)]
        pltpu.async_copy(xr_hbm.at[gidx], rowsa if gidx is gidxa else rowsb,
                         sem)
        pltpu.async_copy(dst_hbm.at[pl.ds(w0 + g * GSA, GSA)], didx, dsem)
        pltpu.async_copy(sc_hbm.at[pl.ds(w0 + g * GSA, GSA)], scl, dsem)

    def _wait(g, gidx, didx, scl, sem, dsem):
        rows = rowsa if gidx is gidxa else rowsb
        pltpu.make_async_copy(xr_hbm.at[gidx], rows, sem).wait()
        pltpu.make_async_copy(dst_hbm.at[pl.ds(w0 + g * GSA, GSA)], didx,
                              dsem).wait()
        pltpu.make_async_copy(sc_hbm.at[pl.ds(w0 + g * GSA, GSA)], scl,
                              dsem).wait()

    def _scale_scatter(rows, didx, scl):
        def _sg(k, _):
            sc16 = scl[pl.ds(k * LN, LN)]
            for j in range(LN):
                scj = sc16[j]
                r = k * LN + j
                for kk in range(H // LN):
                    rows[r, pl.ds(kk * LN, LN)] = (
                        rows[r, pl.ds(kk * LN, LN)] * scj)
            return 0
        lax.fori_loop(0, NGA, _sg, 0)
        pltpu.sync_copy(rows, acc_sh.at[didx], add=True)

    _fire(0, gidxa, didxa, scla, sema, dsema)

    def _pair(k2, _):
        ga = 2 * k2
        _wait(ga, gidxa, didxa, scla, sema, dsema)
        _fire(ga + 1, gidxb, didxb, sclb, semb, dsemb)
        _scale_scatter(rowsa, didxa, scla)
        _wait(ga + 1, gidxb, didxb, sclb, semb, dsemb)
        _fire(ga + 2, gidxa, didxa, scla, sema, dsema)
        _scale_scatter(rowsb, didxb, sclb)
        return 0
    lax.fori_loop(0, (NGP - 1) // 2, _pair, 0)
    _wait(NGP - 1, gidxa, didxa, scla, sema, dsema)
    _scale_scatter(rowsa, didxa, scla)
    plsc.subcore_barrier()

    sb2, sl2 = s * 624, 624
    for j in range(8):
        sz = GSZ if j < 7 else 64
        pltpu.sync_copy(acc_sh.at[pl.ds(sb2 + j * GSZ, sz)],
                        out_hbm.at[c].at[pl.ds(sb2 + j * GSZ, sz)])

    @pl.when(s == NS - 1)
    def _():
        pltpu.sync_copy(acc_sh.at[pl.ds(N - GSZ, GSZ)],
                        out_hbm.at[c].at[pl.ds(N - GSZ, GSZ)])


@functools.lru_cache(maxsize=None)
def _sc_agg_call():
    return pl.kernel(
        _sc_agg_body,
        out_type=jax.ShapeDtypeStruct((NC, N, H), jnp.float32),
        mesh=_sc_mesh(),
        compiler_params=pltpu.CompilerParams(needs_layout_passes=False),
        scratch_types=[
            pltpu.VMEM_SHARED((N, H), jnp.float32),
            pltpu.VMEM((EC32P,), jnp.int32),
            pltpu.VMEM((GSA,), jnp.int32),
            pltpu.VMEM((GSA,), jnp.int32),
            pltpu.VMEM((GSA,), jnp.float32),
            pltpu.VMEM((GSA,), jnp.int32),
            pltpu.VMEM((GSA,), jnp.int32),
            pltpu.VMEM((GSA,), jnp.float32),
            pltpu.VMEM((GSA, H), jnp.float32),
            pltpu.VMEM((GSA, H), jnp.float32),
            pltpu.SemaphoreType.DMA,
            pltpu.SemaphoreType.DMA,
            pltpu.SemaphoreType.DMA,
            pltpu.SemaphoreType.DMA,
        ],
    )


def _pad_edges(x1d):
    return jnp.pad(x1d.reshape(NW, EC32), ((0, 0), (0, EC32P - EC32))).reshape(-1)


def _edge_agg(xr, gidxp, dstp, scalep):
    return _sc_agg_call()(xr.reshape(R * N, H), gidxp, dstp, scalep)


def kernel(nodeTypes, edge_index, edge_attr, bs, sequence_input, nodes_bs,
           len_seq, action_input, emb_nodes_w, emb_actions_w,
           rgcn1_wrel, rgcn1_wroot, rgcn1_b, rgcn2_wrel, rgcn2_wroot, rgcn2_b,
           gru_wih, gru_whh, gru_bih, gru_bhh,
           linAction_w, linAction_b, linActionF_w, linActionF_b,
           finLin_w, finLin_b, finF_w, finF_b,
           linNodes_w, linNodes_b, linNodesF_w, linNodesF_b):
    src1d = edge_index[0].astype(jnp.int32)
    dst1d = edge_index[1].astype(jnp.int32)
    et1d = edge_attr.astype(jnp.int32)

    scale1d, gidx1d = _edge_scale(src1d, dst1d, et1d)
    gidxp = _pad_edges(gidx1d)
    dstp = _pad_edges(dst1d)
    scalep = _pad_edges(scale1d)

    xr1, root1 = _tc1(nodeTypes, emb_nodes_w, rgcn1_wrel, rgcn1_wroot, rgcn1_b)
    agg1 = _edge_agg(xr1, gidxp, dstp, scalep)
    xr2, root2 = _tc2(root1, agg1, rgcn2_wrel, rgcn2_wroot, rgcn2_b)
    agg2 = _edge_agg(xr2, gidxp, dstp, scalep)

    nodeEmb, hg4, mm4 = _tc3(root2, agg2, sequence_input)
    h_G = hg4.reshape(B, H)
    outmm = mm4.reshape(B, L, H)

    w1 = linNodes_w[:, :H]
    w2 = linNodes_w[:, H:]
    action, final, gru_part = _tc4(
        h_G, outmm, action_input, len_seq, emb_actions_w,
        gru_wih, gru_whh, gru_bih, gru_bhh,
        linAction_w, linAction_b, linActionF_w, linActionF_b,
        finLin_w, finLin_b, finF_w, finF_b, w1, linNodes_b)

    probs = _tc5(nodeEmb, gru_part, w2, linNodesF_w)
    nodes_final = probs.reshape(N, L)
    return (action, nodes_final, final)


# speedup vs baseline: 1.4107x; 1.4107x over previous
import functools

import jax
import jax.numpy as jnp
from jax import lax
from jax.experimental import pallas as pl
from jax.experimental.pallas import tpu as pltpu
from jax.experimental.pallas import tpu_sc as plsc

N = 10000
E = 320000
H = 128
R = 8
B = 100
L = 16
VN = 32
VA = 16
G = N // B
NBLK = 10
BN = N // NBLK
GB = B // NBLK


def _tc1_body(nt_ref, emb_ref, wrel_ref, wroot_ref, b_ref, xr_ref, root_ref):
    nt = nt_ref[0]
    iot = lax.broadcasted_iota(jnp.int32, (VN, BN), 0)
    ohT = (iot == nt).astype(jnp.float32)
    emb = emb_ref[...]
    dn = (((0,), (0,)), ((), ()))
    for r in range(R):
        wp = jnp.dot(emb, wrel_ref[r], preferred_element_type=jnp.float32)
        xr_ref[r] = lax.dot_general(ohT, wp, dn, preferred_element_type=jnp.float32)
    wpr = jnp.dot(emb, wroot_ref[...], preferred_element_type=jnp.float32)
    root_ref[...] = (
        lax.dot_general(ohT, wpr, dn, preferred_element_type=jnp.float32) + b_ref[...]
    )


def _tc1(nodeTypes, emb_w, wrel, wroot, b):
    nt3 = nodeTypes.astype(jnp.int32).reshape(NBLK, 1, BN)
    return pl.pallas_call(
        _tc1_body,
        grid=(NBLK,),
        in_specs=[
            pl.BlockSpec((1, 1, BN), lambda i: (i, 0, 0)),
            pl.BlockSpec((VN, H), lambda i: (0, 0)),
            pl.BlockSpec((R, H, H), lambda i: (0, 0, 0)),
            pl.BlockSpec((H, H), lambda i: (0, 0)),
            pl.BlockSpec((1, H), lambda i: (0, 0)),
        ],
        out_specs=[
            pl.BlockSpec((R, BN, H), lambda i: (0, i, 0)),
            pl.BlockSpec((BN, H), lambda i: (i, 0)),
        ],
        out_shape=[
            jax.ShapeDtypeStruct((R, N, H), jnp.float32),
            jax.ShapeDtypeStruct((N, H), jnp.float32),
        ],
    )(nt3, emb_w, wrel, wroot, b.reshape(1, H))


def _tc2_body(root_ref, agg_ref, wrel_ref, wroot_ref, b_ref, xr_ref, root2_ref):
    x = jax.nn.relu(root_ref[...] + agg_ref[0] + agg_ref[1])
    for r in range(R):
        xr_ref[r] = jnp.dot(x, wrel_ref[r], preferred_element_type=jnp.float32)
    root2_ref[...] = (
        jnp.dot(x, wroot_ref[...], preferred_element_type=jnp.float32) + b_ref[...]
    )


def _tc2(root1, agg1, wrel, wroot, b):
    return pl.pallas_call(
        _tc2_body,
        grid=(NBLK,),
        in_specs=[
            pl.BlockSpec((BN, H), lambda i: (i, 0)),
            pl.BlockSpec((2, BN, H), lambda i: (0, i, 0)),
            pl.BlockSpec((R, H, H), lambda i: (0, 0, 0)),
            pl.BlockSpec((H, H), lambda i: (0, 0)),
            pl.BlockSpec((1, H), lambda i: (0, 0)),
        ],
        out_specs=[
            pl.BlockSpec((R, BN, H), lambda i: (0, i, 0)),
            pl.BlockSpec((BN, H), lambda i: (i, 0)),
        ],
        out_shape=[
            jax.ShapeDtypeStruct((R, N, H), jnp.float32),
            jax.ShapeDtypeStruct((N, H), jnp.float32),
        ],
    )(root1, agg1, wrel, wroot, b.reshape(1, H))


def _tc3_body(root_ref, agg_ref, seq_ref, emb_ref, hg_ref, mm_ref):
    embf = jax.nn.relu(root_ref[...] + agg_ref[0] + agg_ref[1])
    emb_ref[...] = embf
    dn = (((0,), (0,)), ((), ()))
    for g in range(GB):
        eg = embf[g * G:(g + 1) * G]
        hg_ref[0, g] = jnp.sum(eg, axis=0) * (1.0 / G)
        sg = seq_ref[0, g]
        mm_ref[0, g] = lax.dot_general(sg, eg, dn, preferred_element_type=jnp.float32)


def _tc3(root2, agg2, seq):
    seq4 = seq.reshape(NBLK, GB, G, L)
    return pl.pallas_call(
        _tc3_body,
        grid=(NBLK,),
        in_specs=[
            pl.BlockSpec((BN, H), lambda i: (i, 0)),
            pl.BlockSpec((2, BN, H), lambda i: (0, i, 0)),
            pl.BlockSpec((1, GB, G, L), lambda i: (i, 0, 0, 0)),
        ],
        out_specs=[
            pl.BlockSpec((BN, H), lambda i: (i, 0)),
            pl.BlockSpec((1, GB, H), lambda i: (i, 0, 0)),
            pl.BlockSpec((1, GB, L, H), lambda i: (i, 0, 0, 0)),
        ],
        out_shape=[
            jax.ShapeDtypeStruct((N, H), jnp.float32),
            jax.ShapeDtypeStruct((NBLK, GB, H), jnp.float32),
            jax.ShapeDtypeStruct((NBLK, GB, L, H), jnp.float32),
        ],
    )(root2, agg2, seq4)


def _tc4_body(hg_ref, mm_ref, act_in_ref, len_ref, embA_ref,
              wih_ref, whh_ref, bih_ref, bhh_ref,
              la_w_ref, la_b_ref, laf_w_ref, laf_b_ref,
              fl_w_ref, fl_b_ref, ff_w_ref, ff_b_ref,
              w1t_ref, lnb_ref,
              action_ref, final_ref, gp_ref):
    h_G = hg_ref[...]
    dn_t = (((1,), (1,)), ((), ()))

    ai = act_in_ref[...]
    iot = lax.broadcasted_iota(jnp.int32, (B, VA), 1)
    oh = (iot == ai).astype(jnp.float32)
    sos = jnp.dot(oh, embA_ref[...], preferred_element_type=jnp.float32)

    wih = wih_ref[...]
    whh = whh_ref[...]
    bih = bih_ref[...]
    bhh = bhh_ref[...]
    lens = len_ref[...]
    w1 = w1t_ref[...]
    lnb = lnb_ref[...]

    h = h_G
    for t in range(L):
        xt = sos if t == 0 else mm_ref[:, t - 1, :]
        gi = lax.dot_general(xt, wih, dn_t, preferred_element_type=jnp.float32) + bih
        gh = lax.dot_general(h, whh, dn_t, preferred_element_type=jnp.float32) + bhh
        i_r, i_z, i_n = gi[:, :H], gi[:, H:2 * H], gi[:, 2 * H:]
        h_r, h_z, h_n = gh[:, :H], gh[:, H:2 * H], gh[:, 2 * H:]
        rr = jax.nn.sigmoid(i_r + h_r)
        zz = jax.nn.sigmoid(i_z + h_z)
        nn_ = jnp.tanh(i_n + rr * h_n)
        h = (1.0 - zz) * nn_ + zz * h
        hm = h * (lens > t).astype(jnp.float32)
        gp_ref[:, t, :] = (
            lax.dot_general(hm, w1, dn_t, preferred_element_type=jnp.float32) + lnb
        )

    hr = jax.nn.relu(
        lax.dot_general(h_G, la_w_ref[...], dn_t, preferred_element_type=jnp.float32)
        + la_b_ref[...])
    action_ref[...] = (
        lax.dot_general(hr, laf_w_ref[...], dn_t, preferred_element_type=jnp.float32)
        + laf_b_ref[...])
    fr = jax.nn.relu(
        lax.dot_general(h_G, fl_w_ref[...], dn_t, preferred_element_type=jnp.float32)
        + fl_b_ref[...])
    s = jnp.sum(fr * ff_w_ref[...], axis=1, keepdims=True)
    final_ref[...] = jax.nn.sigmoid(s + ff_b_ref[...])


def _tc4(h_G, outmm, action_input, len_seq, emb_actions_w,
         gru_wih, gru_whh, gru_bih, gru_bhh,
         linAction_w, linAction_b, linActionF_w, linActionF_b,
         finLin_w, finLin_b, finF_w, finF_b, w1, linNodes_b):
    return pl.pallas_call(
        _tc4_body,
        out_shape=[
            jax.ShapeDtypeStruct((B, VA), jnp.float32),
            jax.ShapeDtypeStruct((B, 1), jnp.float32),
            jax.ShapeDtypeStruct((B, L, H), jnp.float32),
        ],
    )(h_G, outmm, action_input.astype(jnp.int32).reshape(B, 1),
      len_seq.astype(jnp.int32).reshape(B, 1), emb_actions_w,
      gru_wih, gru_whh, gru_bih.reshape(1, 3 * H), gru_bhh.reshape(1, 3 * H),
      linAction_w, linAction_b.reshape(1, H), linActionF_w,
      linActionF_b.reshape(1, VA),
      finLin_w, finLin_b.reshape(1, H), finF_w,
      jnp.broadcast_to(finF_b.reshape(1, 1), (B, 1)),
      w1, linNodes_b.reshape(1, H))


def _tc5_body(emb_ref, gp_ref, w2_ref, f_ref, out_ref):
    w2 = w2_ref[...]
    f = f_ref[...]
    dn_t = (((1,), (1,)), ((), ()))
    for g in range(GB):
        eg = emb_ref[...][g * G:(g + 1) * G]
        npart = lax.dot_general(eg, w2, dn_t, preferred_element_type=jnp.float32)
        gp = gp_ref[g]
        t3 = jax.nn.relu(npart[:, None, :] + gp[None, :, :])
        logits = jnp.sum(t3 * f[0][None, None, :], axis=-1)
        m = jnp.max(logits, axis=0, keepdims=True)
        e = jnp.exp(logits - m)
        s = jnp.sum(e, axis=0, keepdims=True)
        out_ref[0, g] = e / s


def _tc5(nodeEmb, gru_part, w2, f):
    return pl.pallas_call(
        _tc5_body,
        grid=(NBLK,),
        in_specs=[
            pl.BlockSpec((BN, H), lambda i: (i, 0)),
            pl.BlockSpec((GB, L, H), lambda i: (i, 0, 0)),
            pl.BlockSpec((H, H), lambda i: (0, 0)),
            pl.BlockSpec((1, H), lambda i: (0, 0)),
        ],
        out_specs=[pl.BlockSpec((1, GB, G, L), lambda i: (i, 0, 0, 0))],
        out_shape=[jax.ShapeDtypeStruct((NBLK, GB, G, L), jnp.float32)],
    )(nodeEmb, gru_part, w2, f)[0]


NC = 2
NS = 16
NW = NC * NS
LN = 16
GSZ = 80
NGRP = GSZ // LN
EROWS = E // GSZ
EC16 = E // NS
EC32 = E // NW
CH = 25
NSTR = N // NS

@functools.lru_cache(maxsize=None)
def _sc_mesh():
    return plsc.VectorSubcoreMesh(core_axis_name="c", subcore_axis_name="s",
                                  num_cores=NC, num_subcores=NS)


def _iota16():
    return lax.iota(jnp.int32, LN)


def _stripe(s):
    base = s * 624
    length = 624 if s < NS - 1 else N - 624 * (NS - 1)
    return base, length


TROWS = N * R // H
NG = EC32 // GSZ


def _sc_count_body(dst_hbm, et_hbm, out_hbm, tab, dstb, etb):
    c = lax.axis_index("c")
    s = lax.axis_index("s")
    wid = s * NC + c
    zer = jnp.zeros((LN,), jnp.float32)

    def _zt(i, _):
        for kk in range(H // LN):
            tab[i, pl.ds(kk * LN, LN)] = zer
        return 0
    lax.fori_loop(0, TROWS, _zt, 0)

    w0 = wid * EC32
    pltpu.sync_copy(dst_hbm.at[pl.ds(w0, EC32)], dstb)
    pltpu.sync_copy(et_hbm.at[pl.ds(w0, EC32)], etb)
    ones = jnp.ones((LN,), jnp.float32)

    def _grp(i, _):
        idx16 = (dstb[pl.ds(i * LN, LN)] * R + etb[pl.ds(i * LN, LN)])
        plsc.addupdate_scatter(tab, [idx16 >> 7, idx16 & (H - 1)], ones)
        return 0
    lax.fori_loop(0, EC32 // LN, _grp, 0)
    pltpu.sync_copy(tab, out_hbm.at[wid])


@functools.lru_cache(maxsize=None)
def _sc_count_call():
    return pl.kernel(
        _sc_count_body,
        out_type=jax.ShapeDtypeStruct((NW, TROWS, H), jnp.float32),
        mesh=_sc_mesh(),
        compiler_params=pltpu.CompilerParams(needs_layout_passes=False),
        scratch_types=[
            pltpu.VMEM((TROWS, H), jnp.float32),
            pltpu.VMEM((EC32,), jnp.int32),
            pltpu.VMEM((EC32,), jnp.int32),
        ],
    )


def _tc_rcp_body(cnt_ref, src_ref, et_ref, rcp_ref, gidx_ref):
    tot = jnp.sum(cnt_ref[...], axis=0)
    rcp_ref[...] = 1.0 / jnp.maximum(tot, 1.0)
    gidx_ref[...] = et_ref[...] * N + src_ref[...]


def _tc_rcp(cnt_parts, src1d, et1d):
    rcp, gidx = pl.pallas_call(
        _tc_rcp_body,
        out_shape=[
            jax.ShapeDtypeStruct((TROWS, H), jnp.float32),
            jax.ShapeDtypeStruct((E // H, H), jnp.int32),
        ],
    )(cnt_parts, src1d.reshape(E // H, H), et1d.reshape(E // H, H))
    return rcp, gidx.reshape(E)


def _sc_scale2_body(rcp_hbm, dst_hbm, et_hbm, scale_hbm, tab, dstb, etb, scb):
    c = lax.axis_index("c")
    s = lax.axis_index("s")
    wid = s * NC + c
    pltpu.sync_copy(rcp_hbm, tab)
    w0 = wid * EC32
    pltpu.sync_copy(dst_hbm.at[pl.ds(w0, EC32)], dstb)
    pltpu.sync_copy(et_hbm.at[pl.ds(w0, EC32)], etb)

    def _grp(i, _):
        idx16 = (dstb[pl.ds(i * LN, LN)] * R + etb[pl.ds(i * LN, LN)])
        scb[pl.ds(i * LN, LN)] = plsc.load_gather(
            tab, [idx16 >> 7, idx16 & (H - 1)])
        return 0
    lax.fori_loop(0, EC32 // LN, _grp, 0)
    pltpu.sync_copy(scb, scale_hbm.at[pl.ds(w0, EC32)])


@functools.lru_cache(maxsize=None)
def _sc_scale2_call():
    return pl.kernel(
        _sc_scale2_body,
        out_type=jax.ShapeDtypeStruct((E,), jnp.float32),
        mesh=_sc_mesh(),
        compiler_params=pltpu.CompilerParams(needs_layout_passes=False),
        scratch_types=[
            pltpu.VMEM((TROWS, H), jnp.float32),
            pltpu.VMEM((EC32,), jnp.int32),
            pltpu.VMEM((EC32,), jnp.int32),
            pltpu.VMEM((EC32,), jnp.float32),
        ],
    )


def _edge_scale(src1d, dst1d, et1d):
    cnt_parts = _sc_count_call()(dst1d, et1d)
    rcp, gidx1d = _tc_rcp(cnt_parts, src1d, et1d)
    return _sc_scale2_call()(rcp, dst1d, et1d), gidx1d


def _sc_agg_body(xr_hbm, gidx_hbm, dst_hbm, sc_hbm, out_hbm,
                 acc_sh, gsc, sclc,
                 gidxa, didxa, gidxb, didxb, rowsa, rowsb,
                 sema, semb, dsema, dsemb):
    c = lax.axis_index("c")
    s = lax.axis_index("s")
    wid = s * NC + c

    def _zr(i, _):
        for kk in range(H // LN):
            rowsa[i, pl.ds(kk * LN, LN)] = jnp.zeros((LN,), jnp.float32)
        return 0
    lax.fori_loop(0, GSZ, _zr, 0)
    sb = s * 624
    for j in range(8):
        sz = GSZ if j < 7 else 64
        pltpu.sync_copy(rowsa.at[pl.ds(0, sz)],
                        acc_sh.at[pl.ds(sb + j * GSZ, sz)])

    @pl.when(s == NS - 1)
    def _():
        pltpu.sync_copy(rowsa, acc_sh.at[pl.ds(N - GSZ, GSZ)])

    plsc.subcore_barrier()

    w0 = wid * EC32
    pltpu.sync_copy(gidx_hbm.at[pl.ds(w0, EC32)], gsc)
    pltpu.sync_copy(sc_hbm.at[pl.ds(w0, EC32)], sclc)

    def _fire(g, gidx, didx, sem, dsem):
        for k in range(NGRP):
            gidx[pl.ds(k * LN, LN)] = gsc[pl.ds(g * GSZ + k * LN, LN)]
        pltpu.async_copy(xr_hbm.at[gidx], rowsa if gidx is gidxa else rowsb,
                         sem)
        pltpu.async_copy(dst_hbm.at[pl.ds(w0 + g * GSZ, GSZ)], didx, dsem)

    def _wait(g, gidx, didx, sem, dsem):
        rows = rowsa if gidx is gidxa else rowsb
        pltpu.make_async_copy(xr_hbm.at[gidx], rows, sem).wait()
        pltpu.make_async_copy(dst_hbm.at[pl.ds(w0 + g * GSZ, GSZ)], didx,
                              dsem).wait()

    def _scale_scatter(g, rows, didx):
        for k in range(NGRP):
            sc16 = sclc[pl.ds(g * GSZ + k * LN, LN)]
            for j in range(LN):
                scj = sc16[j]
                r = k * LN + j
                for kk in range(H // LN):
                    rows[r, pl.ds(kk * LN, LN)] = (
                        rows[r, pl.ds(kk * LN, LN)] * scj)
        pltpu.sync_copy(rows, acc_sh.at[didx], add=True)

    _fire(0, gidxa, didxa, sema, dsema)

    def _pair(k2, _):
        ga = 2 * k2
        _wait(ga, gidxa, didxa, sema, dsema)
        _fire(ga + 1, gidxb, didxb, semb, dsemb)
        _scale_scatter(ga, rowsa, didxa)
        _wait(ga + 1, gidxb, didxb, semb, dsemb)
        _fire(ga + 2, gidxa, didxa, sema, dsema)
        _scale_scatter(ga + 1, rowsb, didxb)
        return 0
    lax.fori_loop(0, (NG - 1) // 2, _pair, 0)
    _wait(NG - 1, gidxa, didxa, sema, dsema)
    _scale_scatter(NG - 1, rowsa, didxa)
    plsc.subcore_barrier()

    sb2, sl2 = s * 624, 624
    for j in range(8):
        sz = GSZ if j < 7 else 64
        pltpu.sync_copy(acc_sh.at[pl.ds(sb2 + j * GSZ, sz)],
                        out_hbm.at[c].at[pl.ds(sb2 + j * GSZ, sz)])

    @pl.when(s == NS - 1)
    def _():
        pltpu.sync_copy(acc_sh.at[pl.ds(N - GSZ, GSZ)],
                        out_hbm.at[c].at[pl.ds(N - GSZ, GSZ)])


@functools.lru_cache(maxsize=None)
def _sc_agg_call():
    return pl.kernel(
        _sc_agg_body,
        out_type=jax.ShapeDtypeStruct((NC, N, H), jnp.float32),
        mesh=_sc_mesh(),
        compiler_params=pltpu.CompilerParams(needs_layout_passes=False),
        scratch_types=[
            pltpu.VMEM_SHARED((N, H), jnp.float32),
            pltpu.VMEM((EC32,), jnp.int32),
            pltpu.VMEM((EC32,), jnp.float32),
            pltpu.VMEM((GSZ,), jnp.int32),
            pltpu.VMEM((GSZ,), jnp.int32),
            pltpu.VMEM((GSZ,), jnp.int32),
            pltpu.VMEM((GSZ,), jnp.int32),
            pltpu.VMEM((GSZ, H), jnp.float32),
            pltpu.VMEM((GSZ, H), jnp.float32),
            pltpu.SemaphoreType.DMA,
            pltpu.SemaphoreType.DMA,
            pltpu.SemaphoreType.DMA,
            pltpu.SemaphoreType.DMA,
        ],
    )


def _edge_agg(xr, gidx1d, dst1d, scale1d):
    return _sc_agg_call()(xr.reshape(R * N, H), gidx1d, dst1d, scale1d)


def kernel(nodeTypes, edge_index, edge_attr, bs, sequence_input, nodes_bs,
           len_seq, action_input, emb_nodes_w, emb_actions_w,
           rgcn1_wrel, rgcn1_wroot, rgcn1_b, rgcn2_wrel, rgcn2_wroot, rgcn2_b,
           gru_wih, gru_whh, gru_bih, gru_bhh,
           linAction_w, linAction_b, linActionF_w, linActionF_b,
           finLin_w, finLin_b, finF_w, finF_b,
           linNodes_w, linNodes_b, linNodesF_w, linNodesF_b):
    src1d = edge_index[0].astype(jnp.int32)
    dst1d = edge_index[1].astype(jnp.int32)
    et1d = edge_attr.astype(jnp.int32)

    scale1d, gidx1d = _edge_scale(src1d, dst1d, et1d)

    xr1, root1 = _tc1(nodeTypes, emb_nodes_w, rgcn1_wrel, rgcn1_wroot, rgcn1_b)
    agg1 = _edge_agg(xr1, gidx1d, dst1d, scale1d)
    xr2, root2 = _tc2(root1, agg1, rgcn2_wrel, rgcn2_wroot, rgcn2_b)
    agg2 = _edge_agg(xr2, gidx1d, dst1d, scale1d)

    nodeEmb, hg4, mm4 = _tc3(root2, agg2, sequence_input)
    h_G = hg4.reshape(B, H)
    outmm = mm4.reshape(B, L, H)

    w1 = linNodes_w[:, :H]
    w2 = linNodes_w[:, H:]
    action, final, gru_part = _tc4(
        h_G, outmm, action_input, len_seq, emb_actions_w,
        gru_wih, gru_whh, gru_bih, gru_bhh,
        linAction_w, linAction_b, linActionF_w, linActionF_b,
        finLin_w, finLin_b, finF_w, finF_b, w1, linNodes_b)

    probs = _tc5(nodeEmb, gru_part, w2, linNodesF_w)
    nodes_final = probs.reshape(N, L)
    return (action, nodes_final, final)
